# Initial kernel scaffold; baseline (speedup 1.0000x reference)
import jax
import jax.numpy as jnp
from jax.experimental import pallas as pl


def kernel(inputs, W1, b1, g1, be1, W2, b2, g2, be2, Wd, bd):
    raise NotImplementedError("write your pallas kernel here")


# speedup vs baseline: 6.5999x; 6.5999x over previous
import functools

import jax
import jax.numpy as jnp
from jax import lax
from jax.experimental import pallas as pl
from jax.experimental.pallas import tpu as pltpu
from jax.experimental.pallas import tpu_sc as plsc

B, N, D, K = 8, 2048, 128, 20
KPAD = 24
BR = 256


def _knn_body(xr_ref, xf_ref, idx_ref):
    b = pl.program_id(0)
    xr = xr_ref[0]
    xf = xf_ref[0]
    inner = lax.dot_general(xr, xf, (((1,), (1,)), ((), ())),
                            preferred_element_type=jnp.float32)
    sqr = jnp.sum(xr * xr, axis=1, keepdims=True)
    sqf = jnp.sum(xf * xf, axis=1)
    d = sqr - 2.0 * inner + sqf[None, :]
    iota = lax.broadcasted_iota(jnp.int32, (BR, N), 1)
    base = b * N
    idx0 = None
    for k in range(K):
        m = jnp.min(d, axis=1, keepdims=True)
        idxk = jnp.min(jnp.where(d == m, iota, N), axis=1)
        idx_ref[0, k, :] = idxk + base
        if k == 0:
            idx0 = idxk + base
        d = jnp.where(iota == idxk[:, None], jnp.float32(jnp.inf), d)
    for k in range(K, KPAD):
        idx_ref[0, k, :] = idx0


def _knn_tc(x):
    return pl.pallas_call(
        _knn_body,
        grid=(B, N // BR),
        in_specs=[
            pl.BlockSpec((1, BR, D), lambda b, r: (b, r, 0)),
            pl.BlockSpec((1, N, D), lambda b, r: (b, 0, 0)),
        ],
        out_specs=pl.BlockSpec((1, KPAD, BR), lambda b, r: (b, 0, r)),
        out_shape=jax.ShapeDtypeStruct((B, KPAD, N), jnp.int32),
    )(x, x)


_P = 16
_NW = 32
_PPW = (B * N) // _NW
_NBLK = _PPW // _P


def _gather_max_sc(x_flat, idx_pm):
    mesh = plsc.VectorSubcoreMesh(core_axis_name="c", subcore_axis_name="s")

    @functools.partial(
        pl.kernel,
        out_type=jax.ShapeDtypeStruct((B * N, D), jnp.float32),
        mesh=mesh,
        scratch_types=[
            pltpu.VMEM((_P, KPAD), jnp.int32),
            pltpu.VMEM((_P, KPAD, D), jnp.float32),
            pltpu.VMEM((_P, D), jnp.float32),
            pltpu.SemaphoreType.DMA,
        ],
    )
    def k_fn(x_hbm, idx_hbm, out_hbm, idx_v, rows_v, out_v, sem):
        wid = lax.axis_index("s") * 2 + lax.axis_index("c")

        def block(j, _):
            pg = wid * _PPW + j * _P
            pltpu.sync_copy(idx_hbm.at[pl.ds(pg, _P)], idx_v)
            cps = []
            for p in range(_P):
                cp = pltpu.make_async_copy(
                    x_hbm.at[idx_v.at[p]], rows_v.at[p], sem)
                cp.start()
                cps.append(cp)
            for cp in cps:
                cp.wait()

            def row(p, _):
                for dc in range(D // 16):
                    sl = pl.ds(dc * 16, 16)
                    acc = rows_v[p, 0, sl]
                    for k in range(1, KPAD):
                        acc = jnp.maximum(acc, rows_v[p, k, sl])
                    out_v[p, sl] = acc
                return 0

            lax.fori_loop(0, _P, row, 0)
            pltpu.sync_copy(out_v, out_hbm.at[pl.ds(pg, _P)])
            return 0

        lax.fori_loop(0, _NBLK, block, 0)

    return k_fn(x_flat, idx_pm)


def _mlp_body(x_ref, mf_ref, w1a_ref, w1b_ref, b1_ref, g1_ref, be1_ref,
              w2_ref, b2_ref, g2_ref, be2_ref, wd_ref, bd_ref, out_ref):
    eps = 1e-3
    x = x_ref[...]
    mf = mf_ref[...]
    h = lax.dot_general(x, w1a_ref[...], (((1,), (0,)), ((), ())),
                        preferred_element_type=jnp.float32)
    h = h + lax.dot_general(mf - x, w1b_ref[...], (((1,), (0,)), ((), ())),
                            preferred_element_type=jnp.float32)
    h = jnp.maximum(h + b1_ref[...][None, :], 0.0)
    m1 = jnp.mean(h, axis=0, keepdims=True)
    v1 = jnp.mean(jnp.square(h - m1), axis=0, keepdims=True)
    h = g1_ref[...][None, :] * (h - m1) / jnp.sqrt(v1 + eps) + be1_ref[...][None, :]
    h = lax.dot_general(h, w2_ref[...], (((1,), (0,)), ((), ())),
                        preferred_element_type=jnp.float32)
    h = jnp.maximum(h + b2_ref[...][None, :], 0.0)
    m2 = jnp.mean(h, axis=0, keepdims=True)
    v2 = jnp.mean(jnp.square(h - m2), axis=0, keepdims=True)
    h = g2_ref[...][None, :] * (h - m2) / jnp.sqrt(v2 + eps) + be2_ref[...][None, :]
    pooled = jnp.stack(
        [jnp.max(h[bb * N:(bb + 1) * N], axis=0) for bb in range(B)])
    logits = lax.dot_general(pooled, wd_ref[...], (((1,), (0,)), ((), ())),
                             preferred_element_type=jnp.float32)
    logits = logits + bd_ref[...][None, :]
    mx = jnp.max(logits, axis=1, keepdims=True)
    e = jnp.exp(logits - mx)
    out_ref[...] = e / jnp.sum(e, axis=1, keepdims=True)


def _mlp_tc(x_flat, mf, W1a, W1b, b1, g1, be1, W2, b2, g2, be2, Wd, bd):
    return pl.pallas_call(
        _mlp_body,
        out_shape=jax.ShapeDtypeStruct((B, N), jnp.float32),
    )(x_flat, mf, W1a, W1b, b1, g1, be1, W2, b2, g2, be2, Wd, bd)


def kernel(inputs, W1, b1, g1, be1, W2, b2, g2, be2, Wd, bd):
    x = inputs
    idx = _knn_tc(x)
    x_flat = x.reshape(B * N, D)
    idx_pm = jnp.transpose(idx, (0, 2, 1)).reshape(B * N, KPAD)
    mf = _gather_max_sc(x_flat, idx_pm)
    W1a, W1b = W1[:D], W1[D:]
    return _mlp_tc(x_flat, mf, W1a, W1b, b1, g1, be1, W2, b2, g2, be2, Wd, bd)
